# baseline (speedup 1.0000x reference)
import functools

import jax
import jax.numpy as jnp
from jax import lax
from jax.experimental import pallas as pl
from jax.experimental.pallas import tpu as pltpu
from jax.experimental.pallas import tpu_sc as plsc


def kernel(x, pe):
    S, D = pe.shape
    info = plsc.get_sparse_core_info()
    nc, ns = info.num_cores, info.num_subcores
    nw = nc * ns
    rows = S // nw
    CH = 16
    NB = 4
    NCH = rows // CH

    mesh = plsc.VectorSubcoreMesh(core_axis_name="c", subcore_axis_name="s")

    @functools.partial(
        pl.kernel,
        mesh=mesh,
        out_type=jax.ShapeDtypeStruct((S, D), pe.dtype),
        scratch_types=(
            [pltpu.VMEM((CH, D), jnp.float32)] * NB
            + [pltpu.SemaphoreType.DMA] * (2 * NB)
        ),
    )
    def sc_copy(pe_hbm, out_hbm, *scratch):
        bufs = scratch[:NB]
        in_sems = scratch[NB:2 * NB]
        out_sems = scratch[2 * NB:]
        wid = lax.axis_index("s") * nc + lax.axis_index("c")
        base = wid * rows
        in_copies = [None] * NB
        out_copies = [None] * NB

        for c in range(min(NB, NCH)):
            in_copies[c] = pltpu.async_copy(
                pe_hbm.at[pl.ds(base + c * CH, CH)], bufs[c], in_sems[c])
        for c in range(NCH):
            b = c % NB
            in_copies[b].wait()
            out_copies[b] = pltpu.async_copy(
                bufs[b],
                out_hbm.at[pl.ds(base + c * CH, CH)],
                out_sems[b])
            nxt = c + NB
            if nxt < NCH:
                out_copies[b].wait()
                in_copies[b] = pltpu.async_copy(
                    pe_hbm.at[pl.ds(base + nxt * CH, CH)],
                    bufs[b], in_sems[b])
        for b in range(NB):
            if out_copies[b] is not None:
                out_copies[b].wait()

    return sc_copy(pe)[None, :, :]
